# baseline (speedup 1.0000x reference)
import functools

import jax
import jax.numpy as jnp
from jax import lax
from jax.experimental import pallas as pl
from jax.experimental.pallas import tpu as pltpu
from jax.experimental.pallas import tpu_sc as plsc

L = 64
MAXNODE = 512
MAXFANOUT = 8
N0 = 2 * L * MAXNODE * MAXFANOUT
NSEG = N0 // MAXFANOUT
NNZ = 2097152

NC = 2
NS = 16
LANES = 16
NW = NC * NS
EPT = NNZ // NW
CH = 4096
NCHUNK = EPT // CH
G = 128
NG = CH // G
NR = NNZ // G
RPT = EPT // G
RCH = CH // G
O_SL = N0 // NS
T_SL = NSEG // NS

_mesh = functools.partial(
    plsc.VectorSubcoreMesh, core_axis_name="c", subcore_axis_name="s")


def _zero_spm(spm, zbuf, t0, nwords):
    def zb(j, carry):
        zbuf[pl.ds(j * LANES, LANES)] = jnp.zeros((LANES,), jnp.float32)
        return carry
    lax.fori_loop(0, CH // LANES, zb, 0)
    for r in range(nwords // CH):
        pltpu.sync_copy(zbuf, spm.at[pl.ds(t0 + r * CH, CH)])


def _scat_descs(work, spm, i0rb, sem):
    return [pltpu.make_async_copy(work.at[pl.ds(g * G, G)],
                                  spm.at[i0rb.at[g]], sem)
            for g in range(NG)]


def _sc_pass(n_idx, lin_fn, table_words):
    def body(tab_hbm, idx3, val2, out_hbm, tab_spm, acc_spm, *rest):
        nbuf = 2 + n_idx
        set0 = rest[:nbuf] + rest[2 * nbuf:2 * nbuf + 2]
        set1 = rest[nbuf:2 * nbuf] + rest[2 * nbuf + 2:2 * nbuf + 4]
        ssem, gsem, csem = rest[2 * nbuf + 4:]
        sets = [set0, set1]

        c = lax.axis_index("c")
        s = lax.axis_index("s")
        wid = c * NS + s
        t0 = s * O_SL
        tw = table_words // NS
        ts = s * tw
        pltpu.sync_copy(tab_hbm.at[pl.ds(ts, tw)], tab_spm.at[pl.ds(ts, tw)])
        _zero_spm(acc_spm, set0[-1], t0, O_SL)
        plsc.subcore_barrier()

        rbase = wid * RPT

        def stage(pp, ci):
            bb = sets[pp]
            row = pl.multiple_of(lax.min(rbase + ci * RCH, NR - RCH), RCH)
            cps = [pltpu.make_async_copy(idx3.at[k, pl.ds(row, RCH)],
                                         bb[k], ssem)
                   for k in range(1 + n_idx)]
            cps.append(pltpu.make_async_copy(val2.at[pl.ds(row, RCH)],
                                             bb[1 + n_idx], ssem))
            return cps

        def compute_lin(pp):
            ibufs = sets[pp][1:1 + n_idx]
            linb = sets[pp][2 + n_idx]

            def lin_body(j, carry2):
                g = j // (G // LANES)
                cc = pl.ds((j % (G // LANES)) * LANES, LANES)
                linb[pl.ds(j * LANES, LANES)] = lin_fn(
                    [b[g, cc] for b in ibufs])
                return carry2

            lax.fori_loop(0, CH // LANES, lin_body, 0)

        def gather_mul_scat(pp):
            i0rb = sets[pp][0]
            valb = sets[pp][1 + n_idx]
            linb = sets[pp][2 + n_idx]
            work = sets[pp][3 + n_idx]
            gath = [pltpu.make_async_copy(
                        tab_spm.at[linb.at[pl.ds(g * G, G)]],
                        work.at[pl.ds(g * G, G)], gsem)
                    for g in range(NG)]
            for cp in gath:
                cp.start()
            for cp in gath:
                cp.wait()

            def mul_body(j, carry2):
                g = j // (G // LANES)
                cc = pl.ds((j % (G // LANES)) * LANES, LANES)
                sl = pl.ds(j * LANES, LANES)
                work[sl] = valb[g, cc] * work[sl]
                return carry2

            lax.fori_loop(0, CH // LANES, mul_body, 0)
            for cp in _scat_descs(work, acc_spm, i0rb, csem):
                cp.start(add=True)

        def drain_scat(pp):
            i0rb, work = sets[pp][0], sets[pp][3 + n_idx]
            for cp in _scat_descs(work, acc_spm, i0rb, csem):
                cp.wait()

        for cp in stage(0, 0):
            cp.start()

        def chunk_pair(ci2, carry):
            for p in range(2):
                ci = ci2 * 2 + p
                for cp in stage(p, ci):
                    cp.wait()
                compute_lin(p)

                if p == 0:
                    @pl.when(ci2 != 0)
                    def _():
                        drain_scat(1)
                else:
                    drain_scat(0)
                for cp in stage(1 - p, ci + 1):
                    cp.start()
                gather_mul_scat(p)
            return carry

        lax.fori_loop(0, NCHUNK // 2, chunk_pair, 0)
        for cp in stage(0, NCHUNK):
            cp.wait()
        drain_scat(1)
        plsc.subcore_barrier()
        pltpu.sync_copy(acc_spm.at[pl.ds(t0, O_SL)],
                        out_hbm.at[c, pl.ds(t0, O_SL)])

    return body


def _sc_scratch(n_idx, table_words):
    per_set = ([pltpu.VMEM((NG, G), jnp.int32)] * (1 + n_idx) +
               [pltpu.VMEM((NG, G), jnp.float32)])
    tail = [pltpu.VMEM((CH,), jnp.int32), pltpu.VMEM((CH,), jnp.float32)]
    return ([pltpu.VMEM_SHARED((table_words,), jnp.float32),
             pltpu.VMEM_SHARED((N0,), jnp.float32)] +
            per_set + per_set + tail + tail +
            [pltpu.SemaphoreType.DMA] * 3)


def _lin_a(vs):
    v1, v2, v3, v4 = vs
    return ((v1 * L + v2) * MAXNODE + v3) * MAXFANOUT + v4


def _lin_c(vs):
    v1, v2, v3 = vs
    return (v1 * L + v2) * MAXNODE + v3


_pass_a = functools.partial(
    pl.kernel,
    out_type=jax.ShapeDtypeStruct((NC, N0), jnp.float32),
    mesh=_mesh(),
    scratch_types=_sc_scratch(4, N0),
)(_sc_pass(4, _lin_a, N0))

_pass_c = functools.partial(
    pl.kernel,
    out_type=jax.ShapeDtypeStruct((NC, N0), jnp.float32),
    mesh=_mesh(),
    scratch_types=_sc_scratch(3, NSEG),
)(_sc_pass(3, _lin_c, NSEG))


def _tc_sum_body(op_ref, ld_ref, b_ref, o_ref, wl_ref):
    o = op_ref[0] + op_ref[1]
    o_ref[...] = o
    wl_ref[...] = jnp.dot(ld_ref[...] * o, b_ref[...],
                          precision=lax.Precision.HIGHEST,
                          preferred_element_type=jnp.float32)


_tc_sum = pl.pallas_call(
    _tc_sum_body,
    out_shape=[jax.ShapeDtypeStruct((N0 // 128, 128), jnp.float32),
               jax.ShapeDtypeStruct((N0 // 128, 16), jnp.float32)],
)


def _tc_fin_body(wl_ref, o_ref, lwp_ref, b_ref, res_ref):
    lw = lwp_ref[0] + lwp_ref[1]
    res_ref[...] = wl_ref[...] + jnp.dot(
        o_ref[...] * lw, b_ref[...], precision=lax.Precision.HIGHEST,
        preferred_element_type=jnp.float32)


_tc_fin = pl.pallas_call(
    _tc_fin_body,
    out_shape=jax.ShapeDtypeStruct((N0 // 128, 16), jnp.float32),
)


@jax.jit
def kernel(weight, load, adj_indices, adj_values, wire_indices, wire_values):
    w = weight.reshape(-1)
    ld2d = load.reshape(N0 // 128, 128)
    bmat = (jnp.arange(128)[:, None] // MAXFANOUT ==
            jnp.arange(16)[None, :]).astype(jnp.float32)
    o_p = _pass_a(w, adj_indices.reshape(5, NR, G),
                  adj_values.reshape(NR, G))
    o2d, wl16 = _tc_sum(o_p.reshape(NC, N0 // 128, 128), ld2d, bmat)
    lw_p = _pass_c(wl16.reshape(-1), wire_indices.reshape(4, NR, G),
                   wire_values.reshape(NR, G))
    res = _tc_fin(wl16, o2d, lw_p.reshape(NC, N0 // 128, 128), bmat)
    return res.reshape(2, L, MAXNODE)


# speedup vs baseline: 1.2006x; 1.2006x over previous
import functools

import jax
import jax.numpy as jnp
from jax import lax
from jax.experimental import pallas as pl
from jax.experimental.pallas import tpu as pltpu
from jax.experimental.pallas import tpu_sc as plsc

L = 64
MAXNODE = 512
MAXFANOUT = 8
N0 = 2 * L * MAXNODE * MAXFANOUT
NSEG = N0 // MAXFANOUT
NNZ = 2097152

NC = 2
NS = 16
LANES = 16
NW = NC * NS
EPT = NNZ // NW
CH = 2048
NCHUNK = EPT // CH
G = 128
NG = CH // G
O_SL = N0 // NS
T_SL = NSEG // NS

_mesh = functools.partial(
    plsc.VectorSubcoreMesh, core_axis_name="c", subcore_axis_name="s")


def _zero_spm(spm, zbuf, t0, nwords):
    def zb(j, carry):
        zbuf[pl.ds(j * LANES, LANES)] = jnp.zeros((LANES,), jnp.float32)
        return carry
    lax.fori_loop(0, CH // LANES, zb, 0)
    for r in range(nwords // CH):
        pltpu.sync_copy(zbuf, spm.at[pl.ds(t0 + r * CH, CH)])


def _scat_descs(work, spm, i0rb, sem):
    return [pltpu.make_async_copy(work.at[pl.ds(g * G, G)],
                                  spm.at[i0rb.at[g]], sem)
            for g in range(NG)]


def _sc_pass(n_idx, lin_fn, table_words):
    def body(tab_hbm, idxs, vals, out_hbm, tab_spm, acc_spm, *rest):
        set0 = rest[0:5]
        set1 = rest[5:10]
        ssem, gsem, csem = rest[10:]
        sets = [set0, set1]

        c = lax.axis_index("c")
        s = lax.axis_index("s")
        wid = c * NS + s
        t0 = s * O_SL
        tw = table_words // NS
        ts = s * tw
        pltpu.sync_copy(tab_hbm.at[pl.ds(ts, tw)], tab_spm.at[pl.ds(ts, tw)])
        _zero_spm(acc_spm, set0[-1], t0, O_SL)
        plsc.subcore_barrier()

        base = wid * EPT

        def stage(pp, ci):
            ibuf, _, valb, _, _ = sets[pp]
            off = pl.multiple_of(lax.min(base + ci * CH, NNZ - CH), CH)
            return [pltpu.make_async_copy(
                        idxs.at[pl.ds(0, 1 + n_idx), pl.ds(off, CH)],
                        ibuf, ssem),
                    pltpu.make_async_copy(vals.at[pl.ds(off, CH)],
                                          valb, ssem)]

        def compute_lin(pp):
            ibuf, i0rb, _, linb, _ = sets[pp]

            def lin_body(j, carry2):
                g = j // (G // LANES)
                cc = pl.ds((j % (G // LANES)) * LANES, LANES)
                sl = pl.ds(j * LANES, LANES)
                linb[sl] = lin_fn([ibuf[k, sl] for k in range(1, 1 + n_idx)])
                i0rb[g, cc] = ibuf[0, sl]
                return carry2

            lax.fori_loop(0, CH // LANES, lin_body, 0)

        def gather_mul_scat(pp):
            _, i0rb, valb, linb, work = sets[pp]
            gath = [pltpu.make_async_copy(
                        tab_spm.at[linb.at[pl.ds(g * G, G)]],
                        work.at[pl.ds(g * G, G)], gsem)
                    for g in range(NG)]
            for cp in gath:
                cp.start()
            for cp in gath:
                cp.wait()

            def mul_body(j, carry2):
                sl = pl.ds(j * LANES, LANES)
                work[sl] = valb[sl] * work[sl]
                return carry2

            lax.fori_loop(0, CH // LANES, mul_body, 0)
            for cp in _scat_descs(work, acc_spm, i0rb, csem):
                cp.start(add=True)

        def drain_scat(pp):
            i0rb, work = sets[pp][1], sets[pp][4]
            for cp in _scat_descs(work, acc_spm, i0rb, csem):
                cp.wait()

        for cp in stage(0, 0):
            cp.start()

        def chunk_pair(ci2, carry):
            for p in range(2):
                ci = ci2 * 2 + p
                for cp in stage(p, ci):
                    cp.wait()
                compute_lin(p)

                if p == 0:
                    @pl.when(ci2 != 0)
                    def _():
                        drain_scat(1)
                else:
                    drain_scat(0)
                for cp in stage(1 - p, ci + 1):
                    cp.start()
                gather_mul_scat(p)
            return carry

        lax.fori_loop(0, NCHUNK // 2, chunk_pair, 0)
        for cp in stage(0, NCHUNK):
            cp.wait()
        drain_scat(1)
        plsc.subcore_barrier()
        pltpu.sync_copy(acc_spm.at[pl.ds(t0, O_SL)],
                        out_hbm.at[pl.ds(c * N0 + t0, O_SL)])

    return body


def _sc_scratch(n_idx, table_words):
    per_set = [pltpu.VMEM((1 + n_idx, CH), jnp.int32),
               pltpu.VMEM((NG, G), jnp.int32),
               pltpu.VMEM((CH,), jnp.float32),
               pltpu.VMEM((CH,), jnp.int32),
               pltpu.VMEM((CH,), jnp.float32)]
    return ([pltpu.VMEM_SHARED((table_words,), jnp.float32),
             pltpu.VMEM_SHARED((N0,), jnp.float32)] +
            per_set + per_set +
            [pltpu.SemaphoreType.DMA] * 3)


def _lin_a(vs):
    v1, v2, v3, v4 = vs
    return ((v1 * L + v2) * MAXNODE + v3) * MAXFANOUT + v4


def _lin_c(vs):
    v1, v2, v3 = vs
    return (v1 * L + v2) * MAXNODE + v3


_pass_a = functools.partial(
    pl.kernel,
    out_type=jax.ShapeDtypeStruct((NC * N0,), jnp.float32),
    mesh=_mesh(),
    scratch_types=_sc_scratch(4, N0),
)(_sc_pass(4, _lin_a, N0))

_pass_c = functools.partial(
    pl.kernel,
    out_type=jax.ShapeDtypeStruct((NC * N0,), jnp.float32),
    mesh=_mesh(),
    scratch_types=_sc_scratch(3, NSEG),
)(_sc_pass(3, _lin_c, NSEG))


def _tc_sum_body(op_ref, ld_ref, b_ref, o_ref, wl_ref):
    o = op_ref[0] + op_ref[1]
    o_ref[...] = o
    wl_ref[...] = jnp.dot(ld_ref[...] * o, b_ref[...],
                          precision=lax.Precision.HIGHEST,
                          preferred_element_type=jnp.float32)


_tc_sum = pl.pallas_call(
    _tc_sum_body,
    out_shape=[jax.ShapeDtypeStruct((N0 // 128, 128), jnp.float32),
               jax.ShapeDtypeStruct((N0 // 128, 16), jnp.float32)],
)


def _tc_fin_body(wl_ref, o_ref, lwp_ref, b_ref, res_ref):
    lw = lwp_ref[0] + lwp_ref[1]
    res_ref[...] = wl_ref[...] + jnp.dot(
        o_ref[...] * lw, b_ref[...], precision=lax.Precision.HIGHEST,
        preferred_element_type=jnp.float32)


_tc_fin = pl.pallas_call(
    _tc_fin_body,
    out_shape=jax.ShapeDtypeStruct((N0 // 128, 16), jnp.float32),
)


@jax.jit
def kernel(weight, load, adj_indices, adj_values, wire_indices, wire_values):
    w = weight.reshape(-1)
    ld2d = load.reshape(N0 // 128, 128)
    bmat = (jnp.arange(128)[:, None] // MAXFANOUT ==
            jnp.arange(16)[None, :]).astype(jnp.float32)
    o_p = _pass_a(w, adj_indices, adj_values)
    o2d, wl16 = _tc_sum(o_p.reshape(NC, N0 // 128, 128), ld2d, bmat)
    lw_p = _pass_c(wl16.reshape(-1), wire_indices, wire_values)
    res = _tc_fin(wl16, o2d, lw_p.reshape(NC, N0 // 128, 128), bmat)
    return res.reshape(2, L, MAXNODE)


# speedup vs baseline: 1.2460x; 1.0379x over previous
import functools

import jax
import jax.numpy as jnp
from jax import lax
from jax.experimental import pallas as pl
from jax.experimental.pallas import tpu as pltpu
from jax.experimental.pallas import tpu_sc as plsc

L = 64
MAXNODE = 512
MAXFANOUT = 8
N0 = 2 * L * MAXNODE * MAXFANOUT
NSEG = N0 // MAXFANOUT
NNZ = 2097152

NC = 2
NS = 16
LANES = 16
NW = NC * NS
EPT = NNZ // NW
CH = 2048
NCHUNK = EPT // CH
G = 128
NG = CH // G
O_SL = N0 // NS
T_SL = NSEG // NS

_mesh = functools.partial(
    plsc.VectorSubcoreMesh, core_axis_name="c", subcore_axis_name="s")


def _zero_spm(spm, zbuf, t0, nwords, sem):
    def zb(j, carry):
        zbuf[pl.ds(j * LANES, LANES)] = jnp.zeros((LANES,), jnp.float32)
        return carry
    lax.fori_loop(0, CH // LANES, zb, 0)
    cps = [pltpu.make_async_copy(zbuf, spm.at[pl.ds(t0 + r * CH, CH)], sem)
           for r in range(nwords // CH)]
    for cp in cps:
        cp.start()
    for cp in cps:
        cp.wait()


def _scat_descs(work, spm, i0rb, sem):
    return [pltpu.make_async_copy(work.at[pl.ds(g * G, G)],
                                  spm.at[i0rb.at[g]], sem)
            for g in range(NG)]


def _sc_pass(n_idx, lin_fn, table_words):
    def body(tab_hbm, idxs, vals, out_hbm, tab_spm, acc_spm, *rest):
        set0 = rest[0:5]
        set1 = rest[5:10]
        ssem, gsem, csem = rest[10:]
        sets = [set0, set1]

        c = lax.axis_index("c")
        s = lax.axis_index("s")
        wid = c * NS + s
        t0 = s * O_SL
        tw = table_words // NS
        ts = s * tw
        tab_cp = pltpu.make_async_copy(tab_hbm.at[pl.ds(ts, tw)],
                                       tab_spm.at[pl.ds(ts, tw)], gsem)
        tab_cp.start()
        _zero_spm(acc_spm, set0[-1], t0, O_SL, csem)
        tab_cp.wait()
        plsc.subcore_barrier()

        base = wid * EPT

        def stage(pp, ci):
            ibuf, _, valb, _, _ = sets[pp]
            off = pl.multiple_of(lax.min(base + ci * CH, NNZ - CH), CH)
            return [pltpu.make_async_copy(
                        idxs.at[pl.ds(0, 1 + n_idx), pl.ds(off, CH)],
                        ibuf, ssem),
                    pltpu.make_async_copy(vals.at[pl.ds(off, CH)],
                                          valb, ssem)]

        def compute_lin(pp):
            ibuf, i0rb, _, linb, _ = sets[pp]

            def lin_body(j, carry2):
                g = j // (G // LANES)
                cc = pl.ds((j % (G // LANES)) * LANES, LANES)
                sl = pl.ds(j * LANES, LANES)
                linb[sl] = lin_fn([ibuf[k, sl] for k in range(1, 1 + n_idx)])
                i0rb[g, cc] = ibuf[0, sl]
                return carry2

            lax.fori_loop(0, CH // LANES, lin_body, 0)

        def gather_mul_scat(pp):
            _, i0rb, valb, linb, work = sets[pp]
            gath = [pltpu.make_async_copy(
                        tab_spm.at[linb.at[pl.ds(g * G, G)]],
                        work.at[pl.ds(g * G, G)], gsem)
                    for g in range(NG)]
            scat = _scat_descs(work, acc_spm, i0rb, csem)
            for cp in gath:
                cp.start()
            for g in range(NG):
                gath[g].wait()

                def mul_body(j, carry2, base_j=g * (G // LANES)):
                    sl = pl.ds((base_j + j) * LANES, LANES)
                    work[sl] = valb[sl] * work[sl]
                    return carry2

                lax.fori_loop(0, G // LANES, mul_body, 0)
                scat[g].start(add=True)

        def drain_scat(pp):
            i0rb, work = sets[pp][1], sets[pp][4]
            for cp in _scat_descs(work, acc_spm, i0rb, csem):
                cp.wait()

        for cp in stage(0, 0):
            cp.start()

        def chunk_pair(ci2, carry):
            for p in range(2):
                ci = ci2 * 2 + p
                for cp in stage(p, ci):
                    cp.wait()
                compute_lin(p)

                if p == 0:
                    @pl.when(ci2 != 0)
                    def _():
                        drain_scat(1)
                else:
                    drain_scat(0)
                for cp in stage(1 - p, ci + 1):
                    cp.start()
                gather_mul_scat(p)
            return carry

        lax.fori_loop(0, NCHUNK // 2, chunk_pair, 0)
        for cp in stage(0, NCHUNK):
            cp.wait()
        drain_scat(1)
        plsc.subcore_barrier()
        pltpu.sync_copy(acc_spm.at[pl.ds(t0, O_SL)],
                        out_hbm.at[pl.ds(c * N0 + t0, O_SL)])

    return body


def _sc_scratch(n_idx, table_words):
    per_set = [pltpu.VMEM((1 + n_idx, CH), jnp.int32),
               pltpu.VMEM((NG, G), jnp.int32),
               pltpu.VMEM((CH,), jnp.float32),
               pltpu.VMEM((CH,), jnp.int32),
               pltpu.VMEM((CH,), jnp.float32)]
    return ([pltpu.VMEM_SHARED((table_words,), jnp.float32),
             pltpu.VMEM_SHARED((N0,), jnp.float32)] +
            per_set + per_set +
            [pltpu.SemaphoreType.DMA] * 3)


def _lin_a(vs):
    v1, v2, v3, v4 = vs
    return ((v1 * L + v2) * MAXNODE + v3) * MAXFANOUT + v4


def _lin_c(vs):
    v1, v2, v3 = vs
    return (v1 * L + v2) * MAXNODE + v3


_pass_a = functools.partial(
    pl.kernel,
    out_type=jax.ShapeDtypeStruct((NC * N0,), jnp.float32),
    mesh=_mesh(),
    scratch_types=_sc_scratch(4, N0),
)(_sc_pass(4, _lin_a, N0))

_pass_c = functools.partial(
    pl.kernel,
    out_type=jax.ShapeDtypeStruct((NC * N0,), jnp.float32),
    mesh=_mesh(),
    scratch_types=_sc_scratch(3, NSEG),
)(_sc_pass(3, _lin_c, NSEG))


def _tc_sum_body(op_ref, ld_ref, b_ref, o_ref, wl_ref):
    o = op_ref[0] + op_ref[1]
    o_ref[...] = o
    wl_ref[...] = jnp.dot(ld_ref[...] * o, b_ref[...],
                          precision=lax.Precision.HIGHEST,
                          preferred_element_type=jnp.float32)


_tc_sum = pl.pallas_call(
    _tc_sum_body,
    out_shape=[jax.ShapeDtypeStruct((N0 // 128, 128), jnp.float32),
               jax.ShapeDtypeStruct((N0 // 128, 16), jnp.float32)],
)


def _tc_fin_body(wl_ref, o_ref, lwp_ref, b_ref, res_ref):
    lw = lwp_ref[0] + lwp_ref[1]
    res_ref[...] = wl_ref[...] + jnp.dot(
        o_ref[...] * lw, b_ref[...], precision=lax.Precision.HIGHEST,
        preferred_element_type=jnp.float32)


_tc_fin = pl.pallas_call(
    _tc_fin_body,
    out_shape=jax.ShapeDtypeStruct((N0 // 128, 16), jnp.float32),
)


@jax.jit
def kernel(weight, load, adj_indices, adj_values, wire_indices, wire_values):
    w = weight.reshape(-1)
    ld2d = load.reshape(N0 // 128, 128)
    bmat = (jnp.arange(128)[:, None] // MAXFANOUT ==
            jnp.arange(16)[None, :]).astype(jnp.float32)
    o_p = _pass_a(w, adj_indices, adj_values)
    o2d, wl16 = _tc_sum(o_p.reshape(NC, N0 // 128, 128), ld2d, bmat)
    lw_p = _pass_c(wl16.reshape(-1), wire_indices, wire_values)
    res = _tc_fin(wl16, o2d, lw_p.reshape(NC, N0 // 128, 128), bmat)
    return res.reshape(2, L, MAXNODE)
